# Initial kernel scaffold; baseline (speedup 1.0000x reference)
import jax
import jax.numpy as jnp
from jax.experimental import pallas as pl


def kernel(x, edge_index, Wl1, bl1, Wr1, Wl2, bl2, Wr2, gamma, beta):
    raise NotImplementedError("write your pallas kernel here")


# speedup vs baseline: 9.3420x; 9.3420x over previous
import functools

import jax
import jax.numpy as jnp
from jax import lax
from jax.experimental import pallas as pl
from jax.experimental.pallas import tpu as pltpu
from jax.experimental.pallas import tpu_sc as plsc

N = 10000
E = 320000
D_IN = 128
D_H = 256
D_OUT = 128

NC = 2
NS = 16
NW = NC * NS
EPT = E // NW
CHUNK = 125
NCHUNK = EPT // CHUNK
RPS = N // NS

_mesh = plsc.VectorSubcoreMesh(core_axis_name="c", subcore_axis_name="s")


def _make_sc_segsum(with_deg: bool):
    out_type = [jax.ShapeDtypeStruct((NC, N, 128), jnp.float32)]
    scratch = [
        pltpu.VMEM((NCHUNK, CHUNK), jnp.int32),
        pltpu.VMEM((NCHUNK, CHUNK), jnp.int32),
        pltpu.VMEM((CHUNK, 128), jnp.float32),
        pltpu.VMEM_SHARED((N, 128), jnp.float32),
        pltpu.SemaphoreType.DMA,
    ]
    if with_deg:
        out_type.append(jax.ShapeDtypeStruct((NC, N, 16), jnp.float32))
        scratch += [
            pltpu.VMEM((CHUNK, 16), jnp.float32),
            pltpu.VMEM_SHARED((N, 16), jnp.float32),
        ]

    def body(table, srcr, dstr, zrows, *rest):
        if with_deg:
            (zdeg, ones_h, p_out, degp_out, idx_s, idx_d, rows0,
             acc, sem0, ones_v, accd) = rest
        else:
            (p_out, idx_s, idx_d, rows0, acc, sem0) = rest
        c = lax.axis_index("c")
        s = lax.axis_index("s")
        wid = s * NC + c
        r0 = s * RPS
        pltpu.sync_copy(zrows.at[pl.ds(r0, RPS)], acc.at[pl.ds(r0, RPS)])
        if with_deg:
            pltpu.sync_copy(zdeg.at[pl.ds(r0, RPS)], accd.at[pl.ds(r0, RPS)])
            pltpu.sync_copy(ones_h, ones_v)
        pltpu.sync_copy(srcr.at[wid], idx_s)
        pltpu.sync_copy(dstr.at[wid], idx_d)
        plsc.subcore_barrier()

        def step(j, carry):
            pltpu.async_copy(table.at[idx_s.at[j]], rows0, sem0).wait()
            pltpu.sync_copy(rows0, acc.at[idx_d.at[j]], add=True)
            if with_deg:
                pltpu.sync_copy(ones_v, accd.at[idx_d.at[j]], add=True)
            return carry

        lax.fori_loop(0, NCHUNK, step, 0)
        plsc.subcore_barrier()
        pltpu.sync_copy(acc.at[pl.ds(r0, RPS)], p_out.at[c, pl.ds(r0, RPS)])
        if with_deg:
            pltpu.sync_copy(accd.at[pl.ds(r0, RPS)],
                            degp_out.at[c, pl.ds(r0, RPS)])

    return pl.kernel(body, out_type=out_type, mesh=_mesh,
                     scratch_types=scratch,
                     compiler_params=pltpu.CompilerParams(
                         use_tc_tiling_on_sc=False))


_sc_segsum_deg = _make_sc_segsum(with_deg=True)
_sc_segsum = _make_sc_segsum(with_deg=False)

BN = 1000
_GRID = N // BN


def _tc1_body(p0, p1, d0, d1, x, wl, wr, bl, hpre, stats):
    i = pl.program_id(0)
    deg = jnp.maximum(d0[:, 0:1] + d1[:, 0:1], 1.0)
    agg = (p0[...] + p1[...]) / deg
    h = jnp.dot(agg, wl[...], preferred_element_type=jnp.float32)
    h = h + jnp.dot(x[...], wr[...], preferred_element_type=jnp.float32)
    h = h + bl[...]
    hpre[...] = h
    ss = jnp.concatenate([jnp.sum(h, 0, keepdims=True),
                          jnp.sum(h * h, 0, keepdims=True)], axis=0)

    @pl.when(i == 0)
    def _():
        stats[...] = ss

    @pl.when(i != 0)
    def _():
        stats[...] = stats[...] + ss


def _tc2_body(hpre, stats, gamma, beta, wl2, wr2, bl2, p2, r2b):
    st = stats[...]
    mean = st[0:1, :] * (1.0 / N)
    var = st[1:2, :] * (1.0 / N) - mean * mean
    scale = gamma[...] * lax.rsqrt(var + 1e-5)
    h = jnp.maximum((hpre[...] - mean) * scale + beta[...], 0.0)
    p2[...] = jnp.dot(h, wl2[...], preferred_element_type=jnp.float32)
    r2b[...] = (jnp.dot(h, wr2[...], preferred_element_type=jnp.float32)
                + bl2[...])


def _tc3_body(q0, q1, d0, d1, r2b, out):
    deg = jnp.maximum(d0[:, 0:1] + d1[:, 0:1], 1.0)
    out[...] = (q0[...] + q1[...]) / deg + r2b[...]


def _row_spec(w):
    return pl.BlockSpec((BN, w), lambda i: (i, 0))


def _full_spec(shape):
    return pl.BlockSpec(shape, lambda i: tuple(0 for _ in shape))


_tc1 = pl.pallas_call(
    _tc1_body,
    grid=(_GRID,),
    in_specs=[_row_spec(128), _row_spec(128), _row_spec(16), _row_spec(16),
              _row_spec(128), _full_spec((128, 256)), _full_spec((128, 256)),
              _full_spec((1, 256))],
    out_specs=[_row_spec(256), _full_spec((2, 256))],
    out_shape=[jax.ShapeDtypeStruct((N, 256), jnp.float32),
               jax.ShapeDtypeStruct((2, 256), jnp.float32)],
)

_tc2 = pl.pallas_call(
    _tc2_body,
    grid=(_GRID,),
    in_specs=[_row_spec(256), _full_spec((2, 256)), _full_spec((1, 256)),
              _full_spec((1, 256)), _full_spec((256, 128)),
              _full_spec((256, 128)), _full_spec((1, 128))],
    out_specs=[_row_spec(128), _row_spec(128)],
    out_shape=[jax.ShapeDtypeStruct((N, 128), jnp.float32),
               jax.ShapeDtypeStruct((N, 128), jnp.float32)],
)

_tc3 = pl.pallas_call(
    _tc3_body,
    grid=(_GRID,),
    in_specs=[_row_spec(128), _row_spec(128), _row_spec(16), _row_spec(16),
              _row_spec(128)],
    out_specs=_row_spec(128),
    out_shape=jax.ShapeDtypeStruct((N, 128), jnp.float32),
)


def kernel(x, edge_index, Wl1, bl1, Wr1, Wl2, bl2, Wr2, gamma, beta):
    src = edge_index[0].reshape(NW, NCHUNK, CHUNK)
    dst = edge_index[1].reshape(NW, NCHUNK, CHUNK)
    zrows = jnp.zeros((N, 128), jnp.float32)
    zdeg = jnp.zeros((N, 16), jnp.float32)
    ones_h = jnp.ones((CHUNK, 16), jnp.float32)

    P, degP = _sc_segsum_deg(x, src, dst, zrows, zdeg, ones_h)
    d0, d1 = degP[0], degP[1]
    hpre, stats = _tc1(P[0], P[1], d0, d1, x, Wl1, Wr1,
                       bl1.reshape(1, 256))
    p2, r2b = _tc2(hpre, stats, gamma.reshape(1, 256), beta.reshape(1, 256),
                   Wl2, Wr2, bl2.reshape(1, 128))
    (Q,) = _sc_segsum(p2, src, dst, zrows)
    out = _tc3(Q[0], Q[1], d0, d1, r2b)
    return out


# speedup vs baseline: 12.9714x; 1.3885x over previous
import functools

import jax
import jax.numpy as jnp
from jax import lax
from jax.experimental import pallas as pl
from jax.experimental.pallas import tpu as pltpu
from jax.experimental.pallas import tpu_sc as plsc

N = 10000
E = 320000
D_IN = 128
D_H = 256
D_OUT = 128

NC = 2
NS = 16
NW = NC * NS
EPT = E // NW
CHUNK = 100
NCHUNK = EPT // CHUNK
DGRP = 20
RPS = N // NS

_mesh = plsc.VectorSubcoreMesh(core_axis_name="c", subcore_axis_name="s")


def _make_sc_segsum(with_deg: bool):
    out_type = [jax.ShapeDtypeStruct((NC, N, 128), jnp.float32)]
    scratch = [
        pltpu.VMEM((NCHUNK, CHUNK), jnp.int32),
        pltpu.VMEM((DGRP, CHUNK), jnp.int32),
        pltpu.VMEM((CHUNK, 128), jnp.float32),
        pltpu.VMEM((CHUNK, 128), jnp.float32),
        pltpu.VMEM_SHARED((N, 128), jnp.float32),
        pltpu.SemaphoreType.DMA,
        pltpu.SemaphoreType.DMA,
    ]
    if with_deg:
        out_type.append(jax.ShapeDtypeStruct((NC, N, 16), jnp.float32))
        scratch += [
            pltpu.VMEM((CHUNK, 16), jnp.float32),
            pltpu.VMEM_SHARED((N, 16), jnp.float32),
        ]

    def body(table, srcr, dstr, zrows, *rest):
        if with_deg:
            (zdeg, ones_h, p_out, degp_out, idx_s, idx_d, rows0, rows1,
             acc, sem0, sem1, ones_v, accd) = rest
        else:
            (p_out, idx_s, idx_d, rows0, rows1, acc, sem0, sem1) = rest
        c = lax.axis_index("c")
        s = lax.axis_index("s")
        wid = s * NC + c
        r0 = s * RPS
        pltpu.sync_copy(zrows.at[pl.ds(r0, RPS)], acc.at[pl.ds(r0, RPS)])
        if with_deg:
            pltpu.sync_copy(zdeg.at[pl.ds(r0, RPS)], accd.at[pl.ds(r0, RPS)])
            pltpu.sync_copy(ones_h, ones_v)
        pltpu.sync_copy(srcr.at[wid], idx_s)
        plsc.subcore_barrier()

        def gather(j, buf, sem):
            pltpu.async_copy(table.at[idx_s.at[j]], buf, sem)

        def drain_scatter(j_mod, buf, sem):
            pltpu.make_async_copy(table.at[idx_s.at[0]], buf, sem).wait()
            pltpu.sync_copy(buf, acc.at[idx_d.at[j_mod]], add=True)
            if with_deg:
                pltpu.sync_copy(ones_v, accd.at[idx_d.at[j_mod]], add=True)

        gather(0, rows0, sem0)

        def step(t, carry):
            j0 = 2 * t
            g = t // (DGRP // 2)
            jm = j0 % DGRP

            @pl.when(jm == 0)
            def _():
                pltpu.sync_copy(dstr.at[wid, pl.ds(g * DGRP, DGRP)], idx_d)

            gather(j0 + 1, rows1, sem1)
            drain_scatter(jm, rows0, sem0)

            @pl.when(t + 1 < NCHUNK // 2)
            def _():
                gather(j0 + 2, rows0, sem0)

            drain_scatter(jm + 1, rows1, sem1)
            return carry

        lax.fori_loop(0, NCHUNK // 2, step, 0)
        plsc.subcore_barrier()
        pltpu.sync_copy(acc.at[pl.ds(r0, RPS)], p_out.at[c, pl.ds(r0, RPS)])
        if with_deg:
            pltpu.sync_copy(accd.at[pl.ds(r0, RPS)],
                            degp_out.at[c, pl.ds(r0, RPS)])

    return pl.kernel(body, out_type=out_type, mesh=_mesh,
                     scratch_types=scratch,
                     compiler_params=pltpu.CompilerParams(
                         use_tc_tiling_on_sc=False))


_sc_segsum_deg = _make_sc_segsum(with_deg=True)
_sc_segsum = _make_sc_segsum(with_deg=False)

BN = 1000
_GRID = N // BN


def _tc1_body(p0, p1, d0, d1, x, wl, wr, bl, hpre, stats):
    i = pl.program_id(0)
    deg = jnp.maximum(d0[:, 0:1] + d1[:, 0:1], 1.0)
    agg = (p0[...] + p1[...]) / deg
    h = jnp.dot(agg, wl[...], preferred_element_type=jnp.float32)
    h = h + jnp.dot(x[...], wr[...], preferred_element_type=jnp.float32)
    h = h + bl[...]
    hpre[...] = h
    ss = jnp.concatenate([jnp.sum(h, 0, keepdims=True),
                          jnp.sum(h * h, 0, keepdims=True)], axis=0)

    @pl.when(i == 0)
    def _():
        stats[...] = ss

    @pl.when(i != 0)
    def _():
        stats[...] = stats[...] + ss


def _tc2_body(hpre, stats, gamma, beta, wl2, wr2, bl2, p2, r2b):
    st = stats[...]
    mean = st[0:1, :] * (1.0 / N)
    var = st[1:2, :] * (1.0 / N) - mean * mean
    scale = gamma[...] * lax.rsqrt(var + 1e-5)
    h = jnp.maximum((hpre[...] - mean) * scale + beta[...], 0.0)
    p2[...] = jnp.dot(h, wl2[...], preferred_element_type=jnp.float32)
    r2b[...] = (jnp.dot(h, wr2[...], preferred_element_type=jnp.float32)
                + bl2[...])


def _tc3_body(q0, q1, d0, d1, r2b, out):
    deg = jnp.maximum(d0[:, 0:1] + d1[:, 0:1], 1.0)
    out[...] = (q0[...] + q1[...]) / deg + r2b[...]


def _row_spec(w):
    return pl.BlockSpec((BN, w), lambda i: (i, 0))


def _full_spec(shape):
    return pl.BlockSpec(shape, lambda i: tuple(0 for _ in shape))


_tc1 = pl.pallas_call(
    _tc1_body,
    grid=(_GRID,),
    in_specs=[_row_spec(128), _row_spec(128), _row_spec(16), _row_spec(16),
              _row_spec(128), _full_spec((128, 256)), _full_spec((128, 256)),
              _full_spec((1, 256))],
    out_specs=[_row_spec(256), _full_spec((2, 256))],
    out_shape=[jax.ShapeDtypeStruct((N, 256), jnp.float32),
               jax.ShapeDtypeStruct((2, 256), jnp.float32)],
)

_tc2 = pl.pallas_call(
    _tc2_body,
    grid=(_GRID,),
    in_specs=[_row_spec(256), _full_spec((2, 256)), _full_spec((1, 256)),
              _full_spec((1, 256)), _full_spec((256, 128)),
              _full_spec((256, 128)), _full_spec((1, 128))],
    out_specs=[_row_spec(128), _row_spec(128)],
    out_shape=[jax.ShapeDtypeStruct((N, 128), jnp.float32),
               jax.ShapeDtypeStruct((N, 128), jnp.float32)],
)

_tc3 = pl.pallas_call(
    _tc3_body,
    grid=(_GRID,),
    in_specs=[_row_spec(128), _row_spec(128), _row_spec(16), _row_spec(16),
              _row_spec(128)],
    out_specs=_row_spec(128),
    out_shape=jax.ShapeDtypeStruct((N, 128), jnp.float32),
)


def kernel(x, edge_index, Wl1, bl1, Wr1, Wl2, bl2, Wr2, gamma, beta):
    src = edge_index[0].reshape(NW, NCHUNK, CHUNK)
    dst = edge_index[1].reshape(NW, NCHUNK, CHUNK)
    zrows = jnp.zeros((N, 128), jnp.float32)
    zdeg = jnp.zeros((N, 16), jnp.float32)
    ones_h = jnp.ones((CHUNK, 16), jnp.float32)

    P, degP = _sc_segsum_deg(x, src, dst, zrows, zdeg, ones_h)
    d0, d1 = degP[0], degP[1]
    hpre, stats = _tc1(P[0], P[1], d0, d1, x, Wl1, Wr1,
                       bl1.reshape(1, 256))
    p2, r2b = _tc2(hpre, stats, gamma.reshape(1, 256), beta.reshape(1, 256),
                   Wl2, Wr2, bl2.reshape(1, 128))
    (Q,) = _sc_segsum(p2, src, dst, zrows)
    out = _tc3(Q[0], Q[1], d0, d1, r2b)
    return out
